# baseline (speedup 1.0000x reference)
import jax
import jax.numpy as jnp
from jax.experimental import pallas as pl
from jax.experimental.pallas import tpu as pltpu

NA, NH, NW, NCLS, NB = 9, 64, 64, 80, 8
R1 = NA * NH * NW
RW = 128
RROWS = R1 // RW
FROWS = (R1 * 4) // RW
BROWS = NA * NH * 4
NJ = NA
SPL = 4
CG = 32 // SPL
HG = NH // SPL


def _body(awh_ref, bb_ref, cls_ref, xywh_ref, idx_ref, score_ref):
    j = pl.program_id(0)
    s = pl.program_id(1)

    x = cls_ref[...]
    mk = jnp.max(x, axis=-1, keepdims=True)
    score_ref[...] = jnp.swapaxes(jax.nn.sigmoid(mk[..., 0]), 0, 1)
    lane = jax.lax.broadcasted_iota(jnp.int32, x.shape, 3).astype(jnp.float32)
    hit = jnp.where(x == mk, lane, 255.0)
    idx_ref[...] = jnp.swapaxes(jnp.min(hit, axis=-1), 0, 1).astype(jnp.int32)

    fb = bb_ref[...]
    aw = awh_ref[j, 0]
    ah = awh_ref[j, 1]
    ri = jax.lax.broadcasted_iota(jnp.int32, fb.shape, 1)
    wi = jax.lax.broadcasted_iota(jnp.int32, fb.shape, 2)
    c = ri & 3
    h = s * HG + (ri >> 2)
    cx = 4.0 + 8.0 * wi.astype(jnp.float32)
    cy = 4.0 + 8.0 * h.astype(jnp.float32)
    center = jnp.where(c == 0, cx, jnp.where(c == 1, cy, 0.0))
    scale = jnp.where((c & 1) == 0, aw, ah)
    t = jnp.where(c < 2, fb, jnp.exp(fb))
    ov = jnp.clip(center + t * scale, 1.0, 512.0)
    ov = ov.reshape(NB, HG // 2, 8, 64)
    cat = jnp.concatenate([ov[:, :, 0:4, :], ov[:, :, 4:8, :]], axis=3)
    xywh_ref[...] = cat.reshape(NB, 4 * CG, RW)


def kernel(bbox, cls_logits, anchor_wh):
    bbf = bbox.transpose(0, 1, 2, 4, 3).reshape(NB, BROWS, NW)
    cls4 = cls_logits.reshape(NB, RROWS, RW, NCLS)

    xywh, idx, score = pl.pallas_call(
        _body,
        grid=(NJ, SPL),
        compiler_params=pltpu.CompilerParams(
            dimension_semantics=("parallel", "parallel"),
        ),
        in_specs=[
            pl.BlockSpec(memory_space=pltpu.SMEM),
            pl.BlockSpec((NB, 4 * HG, NW), lambda j, s: (0, j * SPL + s, 0)),
            pl.BlockSpec((NB, CG, RW, NCLS), lambda j, s: (0, j * SPL + s, 0, 0)),
        ],
        out_specs=[
            pl.BlockSpec((NB, 4 * CG, RW), lambda j, s: (0, j * SPL + s, 0)),
            pl.BlockSpec((CG, NB, RW), lambda j, s: (j * SPL + s, 0, 0)),
            pl.BlockSpec((CG, NB, RW), lambda j, s: (j * SPL + s, 0, 0)),
        ],
        out_shape=(
            jax.ShapeDtypeStruct((NB, FROWS, RW), jnp.float32),
            jax.ShapeDtypeStruct((RROWS, NB, RW), jnp.int32),
            jax.ShapeDtypeStruct((RROWS, NB, RW), jnp.float32),
        ),
    )(anchor_wh, bbf, cls4)
    xywh = (
        xywh.reshape(NB, RROWS, 4, RW)
        .swapaxes(2, 3)
        .reshape(NB, R1, 4)
    )
    return (
        xywh,
        idx.transpose(1, 0, 2).reshape(NB, R1),
        score.transpose(1, 0, 2).reshape(NB, R1),
    )


# speedup vs baseline: 1.0367x; 1.0367x over previous
import jax
import jax.numpy as jnp
from jax.experimental import pallas as pl
from jax.experimental.pallas import tpu as pltpu

NA, NH, NW, NCLS, NB = 9, 64, 64, 80, 8
R1 = NA * NH * NW
RW = 128
RROWS = R1 // RW
FROWS = (R1 * 4) // RW
BROWS = NA * NH * 4
NJ = NA
SPL = 2
CG = 32 // SPL
HG = NH // SPL


def _body(awh_ref, bb_ref, cls_ref, xywh_ref, idx_ref, score_ref):
    j = pl.program_id(0)
    s = pl.program_id(1)

    x = cls_ref[...]
    mk = jnp.max(x, axis=-1, keepdims=True)
    score_ref[...] = jnp.swapaxes(jax.nn.sigmoid(mk[..., 0]), 0, 1)
    lane = jax.lax.broadcasted_iota(jnp.int32, x.shape, 3).astype(jnp.float32)
    hit = jnp.where(x == mk, lane, 255.0)
    idx_ref[...] = jnp.swapaxes(jnp.min(hit, axis=-1), 0, 1).astype(jnp.int32)

    fb = bb_ref[...]
    aw = awh_ref[0, j]
    ah = awh_ref[1, j]
    ri = jax.lax.broadcasted_iota(jnp.int32, fb.shape, 1)
    wi = jax.lax.broadcasted_iota(jnp.int32, fb.shape, 2)
    c = ri & 3
    h = s * HG + (ri >> 2)
    cx = 4.0 + 8.0 * wi.astype(jnp.float32)
    cy = 4.0 + 8.0 * h.astype(jnp.float32)
    center = jnp.where(c == 0, cx, jnp.where(c == 1, cy, 0.0))
    scale = jnp.where((c & 1) == 0, aw, ah)
    t = jnp.where(c < 2, fb, jnp.exp(fb))
    ov = jnp.clip(center + t * scale, 1.0, 512.0)
    ov = ov.reshape(NB, HG // 2, 8, 64)
    cat = jnp.concatenate([ov[:, :, 0:4, :], ov[:, :, 4:8, :]], axis=3)
    xywh_ref[...] = cat.reshape(NB, 4 * CG, RW)


def kernel(bbox, cls_logits, anchor_wh):
    bbf = bbox.transpose(0, 1, 2, 4, 3).reshape(NB, BROWS, NW)
    cls4 = cls_logits.reshape(NB, RROWS, RW, NCLS)

    xywh, idx, score = pl.pallas_call(
        _body,
        grid=(NJ, SPL),
        compiler_params=pltpu.CompilerParams(
            dimension_semantics=("parallel", "parallel"),
        ),
        in_specs=[
            pl.BlockSpec(memory_space=pltpu.SMEM),
            pl.BlockSpec((NB, 4 * HG, NW), lambda j, s: (0, j * SPL + s, 0)),
            pl.BlockSpec((NB, CG, RW, NCLS), lambda j, s: (0, j * SPL + s, 0, 0)),
        ],
        out_specs=[
            pl.BlockSpec((NB, 4 * CG, RW), lambda j, s: (0, j * SPL + s, 0)),
            pl.BlockSpec((CG, NB, RW), lambda j, s: (j * SPL + s, 0, 0)),
            pl.BlockSpec((CG, NB, RW), lambda j, s: (j * SPL + s, 0, 0)),
        ],
        out_shape=(
            jax.ShapeDtypeStruct((NB, FROWS, RW), jnp.float32),
            jax.ShapeDtypeStruct((RROWS, NB, RW), jnp.int32),
            jax.ShapeDtypeStruct((RROWS, NB, RW), jnp.float32),
        ),
    )(anchor_wh.T, bbf, cls4)
    xywh = (
        xywh.reshape(NB, RROWS, 4, RW)
        .swapaxes(2, 3)
        .reshape(NB, R1, 4)
    )
    return (
        xywh,
        idx.transpose(1, 0, 2).reshape(NB, R1),
        score.transpose(1, 0, 2).reshape(NB, R1),
    )
